# baseline (speedup 1.0000x reference)
import functools

import jax
import jax.numpy as jnp
from jax.experimental import pallas as pl
from jax.experimental.pallas import tpu as pltpu

N = 10000
D_IN = 128
D_OUT = 128
BM = 400


def _gcn_kernel(x_ref, w_ref, b_ref, adj_ref, out_ref, support_ref):
    m = pl.program_id(0)

    @pl.when(m == 0)
    def _():
        support_ref[...] = jnp.dot(
            x_ref[...], w_ref[...], preferred_element_type=jnp.float32
        ).astype(jnp.bfloat16)

    acc = jax.lax.dot_general(
        adj_ref[...], support_ref[...],
        (((1,), (0,)), ((), ())),
        preferred_element_type=jnp.float32,
    )
    out_ref[pl.ds(m * BM, BM), :] = acc + b_ref[...]


@jax.jit
def kernel(input, adj, weight, bias):
    bias2d = bias.reshape(1, D_OUT)
    grid = (N // BM,)
    out = pl.pallas_call(
        _gcn_kernel,
        grid=grid,
        in_specs=[
            pl.BlockSpec((N, D_IN), lambda m: (0, 0)),
            pl.BlockSpec((D_IN, D_OUT), lambda m: (0, 0)),
            pl.BlockSpec((1, D_OUT), lambda m: (0, 0)),
            pl.BlockSpec((BM, N), lambda m: (m, 0)),
        ],
        out_specs=pl.BlockSpec((N, D_OUT), lambda m: (0, 0)),
        out_shape=jax.ShapeDtypeStruct((N, D_OUT), jnp.float32),
        scratch_shapes=[pltpu.VMEM((N, D_OUT), jnp.bfloat16)],
        compiler_params=pltpu.CompilerParams(
            dimension_semantics=("arbitrary",),
        ),
    )(input, weight, bias2d, adj)
    return out


# speedup vs baseline: 1.0050x; 1.0050x over previous
import functools

import jax
import jax.numpy as jnp
from jax.experimental import pallas as pl
from jax.experimental.pallas import tpu as pltpu

N = 10000
D_IN = 128
D_OUT = 128
BM = 400


def _gcn_kernel(x_ref, w_ref, b_ref, adj_ref, out_ref):
    m = pl.program_id(0)
    t = jnp.dot(adj_ref[...], x_ref[...], preferred_element_type=jnp.float32)
    acc = jnp.dot(t, w_ref[...], preferred_element_type=jnp.float32)
    out_ref[pl.ds(m * BM, BM), :] = acc + b_ref[...]


@jax.jit
def kernel(input, adj, weight, bias):
    bias2d = bias.reshape(1, D_OUT)
    grid = (N // BM,)
    out = pl.pallas_call(
        _gcn_kernel,
        grid=grid,
        in_specs=[
            pl.BlockSpec((N, D_IN), lambda m: (0, 0)),
            pl.BlockSpec((D_IN, D_OUT), lambda m: (0, 0)),
            pl.BlockSpec((1, D_OUT), lambda m: (0, 0)),
            pl.BlockSpec((BM, N), lambda m: (m, 0)),
        ],
        out_specs=pl.BlockSpec((N, D_OUT), lambda m: (0, 0)),
        out_shape=jax.ShapeDtypeStruct((N, D_OUT), jnp.float32),
        compiler_params=pltpu.CompilerParams(
            dimension_semantics=("arbitrary",),
        ),
    )(input, weight, bias2d, adj)
    return out
